# baseline (speedup 1.0000x reference)
import functools
import jax
import jax.numpy as jnp
from jax import lax
from jax.experimental import pallas as pl
from jax.experimental.pallas import tpu as pltpu
from jax.experimental.pallas import tpu_sc as plsc

NC, NS, L = 2, 16, 16
NW = NC * NS
N_NODES = 10000
HID = 16
CH = 128
NPAD = 10112
RPT = NPAD // NS
NBUF = 8

_mesh = plsc.VectorSubcoreMesh(core_axis_name="c", subcore_axis_name="s",
                               num_cores=NC, num_subcores=NS)


def _zero_rows(buf, nrows):
    def body(i, _):
        buf[i, :] = jnp.zeros((L,), jnp.float32)
        return 0
    lax.fori_loop(0, nrows, body, 0)


def _rsqrt16(x):
    i = plsc.bitcast(x, jnp.int32)
    i = 0x5F3759DF - lax.shift_right_logical(i, 1)
    y = plsc.bitcast(i, jnp.float32)
    for _ in range(3):
        y = y * (1.5 - 0.5 * x * y * y)
    return y


def _make_deg_kernel(nchunk):
    @functools.partial(
        pl.kernel,
        out_type=jax.ShapeDtypeStruct((NC, NPAD, L), jnp.float32),
        mesh=_mesh,
        scratch_types=[
            pltpu.VMEM((nchunk, CH), jnp.int32),
            pltpu.VMEM((CH, L), jnp.float32),
            pltpu.VMEM((RPT, L), jnp.float32),
            pltpu.VMEM_SHARED((NPAD, L), jnp.float32),
            pltpu.SemaphoreType.DMA,
        ],
        compiler_params=pltpu.CompilerParams(use_tc_tiling_on_sc=False, needs_layout_passes=False),
    )
    def deg_kernel(dst_hbm, out_hbm, dst_v, ones_v, zbuf, acc, sem):
        c = lax.axis_index("c")
        s = lax.axis_index("s")
        wid = c * NS + s
        _zero_rows(zbuf, RPT)
        pltpu.sync_copy(zbuf, acc.at[pl.ds(s * RPT, RPT)])

        def fill_ones(i, _):
            ones_v[i, :] = jnp.ones((L,), jnp.float32)
            return 0
        lax.fori_loop(0, CH, fill_ones, 0)
        pltpu.sync_copy(dst_hbm.at[wid], dst_v)
        plsc.subcore_barrier()

        def fire(j, _):
            pltpu.async_copy(ones_v, acc.at[dst_v.at[j]], sem, add=True)
            return 0
        lax.fori_loop(0, nchunk, fire, 0)

        def drain(j, _):
            pltpu.make_async_copy(ones_v, acc.at[dst_v.at[j]], sem).wait()
            return 0
        lax.fori_loop(0, nchunk, drain, 0)
        plsc.subcore_barrier()
        pltpu.sync_copy(acc.at[pl.ds(s * RPT, RPT)],
                        out_hbm.at[c, pl.ds(s * RPT, RPT)])

    return deg_kernel


def _agg_core(nchunk, src_v, dst_v, gbuf, tbl, acc, gsem, ssem):
    def gissue(j, slot):
        pltpu.async_copy(tbl.at[src_v.at[j]], gbuf.at[slot], gsem.at[slot])

    def gwait(j, slot):
        pltpu.make_async_copy(tbl.at[src_v.at[j]], gbuf.at[slot],
                              gsem.at[slot]).wait()

    def sissue(j, slot):
        pltpu.async_copy(gbuf.at[slot], acc.at[dst_v.at[j]], ssem.at[slot],
                         add=True)

    def swait(j, slot):
        pltpu.make_async_copy(gbuf.at[slot], acc.at[dst_v.at[j]],
                              ssem.at[slot]).wait()

    la = NBUF // 2
    for j0 in range(la):
        gissue(j0, j0)
    ngroups = (nchunk + NBUF - 1) // NBUF

    def group(g, _):
        base = g * NBUF
        for b in range(NBUF):
            j = base + b
            b2 = (b + la) % NBUF

            @pl.when(j < nchunk)
            def _():
                gwait(j, b)
                sissue(j, b)

                @pl.when(j >= la)
                def _():
                    swait(j - la, b2)

                @pl.when(j + la < nchunk)
                def _():
                    gissue(j + la, b2)
        return 0

    lax.fori_loop(0, ngroups, group, 0)
    for k in range(la):
        j = nchunk - la + k
        swait(j, j % NBUF)


def _make_agg1_kernel(nchunk):
    @functools.partial(
        pl.kernel,
        out_type=[
            jax.ShapeDtypeStruct((NC, NPAD, L), jnp.float32),
            jax.ShapeDtypeStruct((NPAD, L), jnp.float32),
            jax.ShapeDtypeStruct((NPAD, L), jnp.float32),
        ],
        mesh=_mesh,
        scratch_types=[
            pltpu.VMEM((nchunk, CH), jnp.int32),
            pltpu.VMEM((nchunk, CH), jnp.int32),
            pltpu.VMEM((NBUF, CH, L), jnp.float32),
            pltpu.VMEM((RPT, L), jnp.float32),
            pltpu.VMEM((RPT, L), jnp.float32),
            pltpu.VMEM((RPT, L), jnp.float32),
            pltpu.VMEM((RPT, L), jnp.float32),
            pltpu.VMEM_SHARED((NPAD, L), jnp.float32),
            pltpu.VMEM_SHARED((NPAD, L), jnp.float32),
            pltpu.SemaphoreType.DMA((NBUF,)),
            pltpu.SemaphoreType.DMA((NBUF,)),
        ],
        compiler_params=pltpu.CompilerParams(use_tc_tiling_on_sc=False, needs_layout_passes=False),
    )
    def agg1_kernel(h_hbm, deg_hbm, src_hbm, dst_hbm,
                    out_hbm, g1_hbm, dis_hbm,
                    src_v, dst_v, gbuf, hv, av, bv, zbuf, tbl, acc,
                    gsem, ssem):
        c = lax.axis_index("c")
        s = lax.axis_index("s")
        wid = c * NS + s
        r0 = s * RPT
        _zero_rows(zbuf, RPT)
        c0 = pltpu.async_copy(zbuf, acc.at[pl.ds(r0, RPT)], ssem.at[0])
        c1 = pltpu.async_copy(h_hbm.at[pl.ds(r0, RPT)], hv, gsem.at[0])
        c2 = pltpu.async_copy(deg_hbm.at[0, pl.ds(r0, RPT)], av, gsem.at[1])
        c3 = pltpu.async_copy(deg_hbm.at[1, pl.ds(r0, RPT)], bv, gsem.at[2])
        c4 = pltpu.async_copy(src_hbm.at[wid], src_v, gsem.at[3])
        c5 = pltpu.async_copy(dst_hbm.at[wid], dst_v, ssem.at[1])
        c1.wait()
        c2.wait()
        c3.wait()

        def build(i, _):
            for u in range(4):
                r = i * 4 + u
                cnt = av[r, :] + bv[r, :] + 1.0
                dis = _rsqrt16(cnt)
                av[r, :] = dis
                bv[r, :] = dis * hv[r, :]
            return 0
        lax.fori_loop(0, RPT // 4, build, 0)
        pltpu.sync_copy(bv, tbl.at[pl.ds(r0, RPT)])

        @pl.when(c == 0)
        def _():
            pltpu.sync_copy(bv, g1_hbm.at[pl.ds(r0, RPT)])
            pltpu.sync_copy(av, dis_hbm.at[pl.ds(r0, RPT)])
        c0.wait()
        c4.wait()
        c5.wait()
        plsc.subcore_barrier()

        _agg_core(nchunk, src_v, dst_v, gbuf, tbl, acc, gsem, ssem)
        plsc.subcore_barrier()
        pltpu.sync_copy(acc.at[pl.ds(r0, RPT)],
                        out_hbm.at[c, pl.ds(r0, RPT)])

    return agg1_kernel


def _make_agg2_kernel(nchunk):
    @functools.partial(
        pl.kernel,
        out_type=[
            jax.ShapeDtypeStruct((NC, NPAD, L), jnp.float32),
            jax.ShapeDtypeStruct((NPAD, L), jnp.float32),
        ],
        mesh=_mesh,
        scratch_types=[
            pltpu.VMEM((nchunk, CH), jnp.int32),
            pltpu.VMEM((nchunk, CH), jnp.int32),
            pltpu.VMEM((NBUF, CH, L), jnp.float32),
            pltpu.VMEM((RPT, L), jnp.float32),
            pltpu.VMEM((RPT, L), jnp.float32),
            pltpu.VMEM((RPT, L), jnp.float32),
            pltpu.VMEM((RPT, L), jnp.float32),
            pltpu.VMEM((RPT, L), jnp.float32),
            pltpu.VMEM_SHARED((NPAD, L), jnp.float32),
            pltpu.VMEM_SHARED((NPAD, L), jnp.float32),
            pltpu.SemaphoreType.DMA((NBUF,)),
            pltpu.SemaphoreType.DMA((NBUF,)),
        ],
        compiler_params=pltpu.CompilerParams(use_tc_tiling_on_sc=False, needs_layout_passes=False),
    )
    def agg2_kernel(g1_hbm, p_hbm, dis_hbm, b1_hbm, src_hbm, dst_hbm,
                    out_hbm, q_hbm,
                    src_v, dst_v, gbuf, gv, av, bv, dv, zbuf, tbl, acc,
                    gsem, ssem):
        c = lax.axis_index("c")
        s = lax.axis_index("s")
        wid = c * NS + s
        r0 = s * RPT
        _zero_rows(zbuf, RPT)
        c0 = pltpu.async_copy(zbuf, acc.at[pl.ds(r0, RPT)], ssem.at[0])
        c1 = pltpu.async_copy(g1_hbm.at[pl.ds(r0, RPT)], gv, gsem.at[0])
        c2 = pltpu.async_copy(p_hbm.at[0, pl.ds(r0, RPT)], av, gsem.at[1])
        c3 = pltpu.async_copy(p_hbm.at[1, pl.ds(r0, RPT)], bv, gsem.at[2])
        c4 = pltpu.async_copy(dis_hbm.at[pl.ds(r0, RPT)], dv, gsem.at[3])
        c5 = pltpu.async_copy(src_hbm.at[wid], src_v, ssem.at[1])
        c6 = pltpu.async_copy(dst_hbm.at[wid], dst_v, ssem.at[2])
        pltpu.sync_copy(b1_hbm, zbuf.at[pl.ds(0, 1)])
        b1 = zbuf[0, :]
        c1.wait()
        c2.wait()
        c3.wait()
        c4.wait()

        def build(i, _):
            for u in range(4):
                r = i * 4 + u
                dis = dv[r, :]
                a = dis * (av[r, :] + bv[r, :] + gv[r, :]) + b1
                bv[r, :] = dis * jnp.maximum(a, 0.0)
            return 0
        lax.fori_loop(0, RPT // 4, build, 0)
        pltpu.sync_copy(bv, tbl.at[pl.ds(r0, RPT)])

        @pl.when(c == 0)
        def _():
            pltpu.sync_copy(bv, q_hbm.at[pl.ds(r0, RPT)])
        c0.wait()
        c5.wait()
        c6.wait()
        plsc.subcore_barrier()

        _agg_core(nchunk, src_v, dst_v, gbuf, tbl, acc, gsem, ssem)
        plsc.subcore_barrier()
        pltpu.sync_copy(acc.at[pl.ds(r0, RPT)],
                        out_hbm.at[c, pl.ds(r0, RPT)])

    return agg2_kernel


def _matmul_kernel(x_ref, w_ref, h_ref):
    h_ref[...] = jnp.dot(x_ref[...], w_ref[...],
                         preferred_element_type=jnp.float32)


def _final_kernel(p_ref, q_ref, dis_ref, w_ref, b_ref, o_ref):
    n = o_ref.shape[0]
    s = p_ref[0, :n, :] + p_ref[1, :n, :]
    o = dis_ref[:n, :] * (s + q_ref[:n, :])
    o_ref[...] = (jnp.dot(o, w_ref[...], preferred_element_type=jnp.float32)
                  + b_ref[...])


def kernel(x, edge_index, W1, b1, W2, b2):
    n, f_in = x.shape
    hid = W1.shape[1]
    c_out = W2.shape[1]
    e = edge_index.shape[1]

    per_dma = NW * CH
    nchunk = -(-e // per_dma)
    ep = nchunk * per_dma
    src = edge_index[0]
    dst = edge_index[1]
    padv = (n + jnp.arange(ep - e, dtype=jnp.int32) % (NPAD - n)
            ).astype(jnp.int32)
    srcw = jnp.concatenate([src, padv]).reshape(NW, nchunk, CH)
    dstw = jnp.concatenate([dst, padv]).reshape(NW, nchunk, CH)

    deg_k = _make_deg_kernel(nchunk)
    agg1_k = _make_agg1_kernel(nchunk)
    agg2_k = _make_agg2_kernel(nchunk)

    deg_parts = deg_k(dstw)

    mblk = 1000
    grid = (n // mblk,)
    h = pl.pallas_call(
        _matmul_kernel,
        grid=grid,
        in_specs=[
            pl.BlockSpec((mblk, f_in), lambda i: (i, 0)),
            pl.BlockSpec((f_in, hid), lambda i: (0, 0)),
        ],
        out_specs=pl.BlockSpec((mblk, hid), lambda i: (i, 0)),
        out_shape=jax.ShapeDtypeStruct((n, hid), jnp.float32),
    )(x, W1)
    hp = jnp.pad(h, ((0, NPAD - n), (0, 0)))

    s1_parts, g1p, dis_tp = agg1_k(hp, deg_parts, srcw, dstw)

    s2_parts, qp = agg2_k(g1p, s1_parts, dis_tp,
                          jnp.broadcast_to(b1, (1, hid)), srcw, dstw)

    outp = pl.pallas_call(
        _final_kernel,
        out_shape=jax.ShapeDtypeStruct((n, c_out), jnp.float32),
    )(s2_parts, qp, dis_tp, W2, jnp.broadcast_to(b2, (1, c_out)))
    return outp
